# baseline (speedup 1.0000x reference)
import jax
import jax.numpy as jnp
from jax.experimental import pallas as pl
from jax.experimental.pallas import tpu as pltpu

NUM_TOKENS = 8192
HIDDEN = 2048
NUM_EXPERTS = 64
TOP_K = 2
BLOCK_T = 512


def _router_block(x_ref, w_ref, w_out_ref, idx_out_ref, aux_ref,
                  cnt_ref, psum_ref):
    i = pl.program_id(0)

    @pl.when(i == 0)
    def _init():
        cnt_ref[:] = jnp.zeros_like(cnt_ref)
        psum_ref[:] = jnp.zeros_like(psum_ref)

    logits = jax.lax.dot_general(
        x_ref[:], w_ref[:], (((1,), (1,)), ((), ())),
        preferred_element_type=jnp.float32)
    m = jnp.max(logits, axis=-1, keepdims=True)
    e = jnp.exp(logits - m)
    s = jnp.sum(e, axis=-1, keepdims=True)
    inv_s = 1.0 / s
    probs = e * inv_s

    iota = jax.lax.broadcasted_iota(jnp.int32, logits.shape, 1)
    idx1 = jnp.argmax(logits, axis=-1, keepdims=True).astype(jnp.int32)
    onehot1 = iota == idx1
    neg_inf = jnp.float32(-jnp.inf)
    logits_m = jnp.where(onehot1, neg_inf, logits)
    l2 = jnp.max(logits_m, axis=-1, keepdims=True)
    idx2 = jnp.argmax(logits_m, axis=-1, keepdims=True).astype(jnp.int32)

    p1 = inv_s
    p2 = jnp.exp(l2 - m) * inv_s
    denom = p1 + p2 + 1e-9
    w_out_ref[:, 0:1] = p1 / denom
    w_out_ref[:, 1:2] = p2 / denom
    idx_out_ref[:, 0:1] = idx1
    idx_out_ref[:, 1:2] = idx2

    cnt_ref[:] += jnp.sum(onehot1.astype(jnp.float32), axis=0, keepdims=True)
    psum_ref[:] += jnp.sum(probs, axis=0, keepdims=True)

    @pl.when(i == pl.num_programs(0) - 1)
    def _finalize():
        scale = NUM_EXPERTS / (NUM_TOKENS * NUM_TOKENS)
        aux_ref[0] = scale * jnp.sum(cnt_ref[:] * psum_ref[:])


def kernel(x, W):
    grid = NUM_TOKENS // BLOCK_T
    w_out, idx_out, aux = pl.pallas_call(
        _router_block,
        grid=(grid,),
        in_specs=[
            pl.BlockSpec((BLOCK_T, HIDDEN), lambda i: (i, 0)),
            pl.BlockSpec((NUM_EXPERTS, HIDDEN), lambda i: (0, 0)),
        ],
        out_specs=[
            pl.BlockSpec((BLOCK_T, TOP_K), lambda i: (i, 0)),
            pl.BlockSpec((BLOCK_T, TOP_K), lambda i: (i, 0)),
            pl.BlockSpec(memory_space=pltpu.SMEM),
        ],
        out_shape=[
            jax.ShapeDtypeStruct((NUM_TOKENS, TOP_K), jnp.float32),
            jax.ShapeDtypeStruct((NUM_TOKENS, TOP_K), jnp.int32),
            jax.ShapeDtypeStruct((1,), jnp.float32),
        ],
        scratch_shapes=[
            pltpu.VMEM((1, NUM_EXPERTS), jnp.float32),
            pltpu.VMEM((1, NUM_EXPERTS), jnp.float32),
        ],
    )(x, W)
    return (w_out, idx_out, aux[0])


# speedup vs baseline: 1.1073x; 1.1073x over previous
import jax
import jax.numpy as jnp
from jax.experimental import pallas as pl
from jax.experimental.pallas import tpu as pltpu

NUM_TOKENS = 8192
HIDDEN = 2048
NUM_EXPERTS = 64
TOP_K = 2
BLOCK_T = 2048


def _router_block(x_ref, w_ref, w_out_ref, idx_out_ref, aux_ref,
                  cnt_ref, psum_ref):
    i = pl.program_id(0)

    @pl.when(i == 0)
    def _init():
        cnt_ref[:] = jnp.zeros_like(cnt_ref)
        psum_ref[:] = jnp.zeros_like(psum_ref)

    logits = jax.lax.dot_general(
        x_ref[:], w_ref[:], (((1,), (1,)), ((), ())),
        preferred_element_type=jnp.float32)
    m = jnp.max(logits, axis=-1, keepdims=True)
    e = jnp.exp(logits - m)
    s = jnp.sum(e, axis=-1, keepdims=True)
    inv_s = 1.0 / s
    probs = e * inv_s

    iota = jax.lax.broadcasted_iota(jnp.int32, logits.shape, 1)
    idx1 = jnp.argmax(logits, axis=-1, keepdims=True).astype(jnp.int32)
    onehot1 = iota == idx1
    neg_inf = jnp.float32(-jnp.inf)
    logits_m = jnp.where(onehot1, neg_inf, logits)
    l2 = jnp.max(logits_m, axis=-1, keepdims=True)
    idx2 = jnp.argmax(logits_m, axis=-1, keepdims=True).astype(jnp.int32)

    p1 = inv_s
    p2 = jnp.exp(l2 - m) * inv_s
    denom = p1 + p2 + 1e-9
    w_out_ref[:, 0:1] = p1 / denom
    w_out_ref[:, 1:2] = p2 / denom
    idx_out_ref[:, 0:1] = idx1
    idx_out_ref[:, 1:2] = idx2

    cnt_ref[:] += jnp.sum(onehot1.astype(jnp.float32), axis=0, keepdims=True)
    psum_ref[:] += jnp.sum(probs, axis=0, keepdims=True)

    @pl.when(i == pl.num_programs(0) - 1)
    def _finalize():
        scale = NUM_EXPERTS / (NUM_TOKENS * NUM_TOKENS)
        aux_ref[0] = scale * jnp.sum(cnt_ref[:] * psum_ref[:])


def kernel(x, W):
    grid = NUM_TOKENS // BLOCK_T
    w_out, idx_out, aux = pl.pallas_call(
        _router_block,
        grid=(grid,),
        in_specs=[
            pl.BlockSpec((BLOCK_T, HIDDEN), lambda i: (i, 0)),
            pl.BlockSpec((NUM_EXPERTS, HIDDEN), lambda i: (0, 0)),
        ],
        out_specs=[
            pl.BlockSpec((BLOCK_T, TOP_K), lambda i: (i, 0)),
            pl.BlockSpec((BLOCK_T, TOP_K), lambda i: (i, 0)),
            pl.BlockSpec(memory_space=pltpu.SMEM),
        ],
        out_shape=[
            jax.ShapeDtypeStruct((NUM_TOKENS, TOP_K), jnp.float32),
            jax.ShapeDtypeStruct((NUM_TOKENS, TOP_K), jnp.int32),
            jax.ShapeDtypeStruct((1,), jnp.float32),
        ],
        scratch_shapes=[
            pltpu.VMEM((1, NUM_EXPERTS), jnp.float32),
            pltpu.VMEM((1, NUM_EXPERTS), jnp.float32),
        ],
    )(x, W)
    return (w_out, idx_out, aux[0])


# speedup vs baseline: 1.1164x; 1.0082x over previous
import jax
import jax.numpy as jnp
from jax.experimental import pallas as pl
from jax.experimental.pallas import tpu as pltpu

NUM_TOKENS = 8192
HIDDEN = 2048
NUM_EXPERTS = 64
TOP_K = 2
BLOCK_T = 1024


def _router_block(x_ref, w_ref, w_out_ref, idx_out_ref, aux_ref,
                  cnt_ref, psum_ref):
    i = pl.program_id(0)

    @pl.when(i == 0)
    def _init():
        cnt_ref[:] = jnp.zeros_like(cnt_ref)
        psum_ref[:] = jnp.zeros_like(psum_ref)

    logits = jax.lax.dot_general(
        x_ref[:], w_ref[:], (((1,), (1,)), ((), ())),
        preferred_element_type=jnp.float32)
    m = jnp.max(logits, axis=-1, keepdims=True)
    e = jnp.exp(logits - m)
    s = jnp.sum(e, axis=-1, keepdims=True)
    inv_s = 1.0 / s
    probs = e * inv_s

    iota = jax.lax.broadcasted_iota(jnp.int32, logits.shape, 1)
    idx1 = jnp.argmax(logits, axis=-1, keepdims=True).astype(jnp.int32)
    onehot1 = iota == idx1
    neg_inf = jnp.float32(-jnp.inf)
    logits_m = jnp.where(onehot1, neg_inf, logits)
    l2 = jnp.max(logits_m, axis=-1, keepdims=True)
    idx2 = jnp.argmax(logits_m, axis=-1, keepdims=True).astype(jnp.int32)

    p1 = inv_s
    p2 = jnp.exp(l2 - m) * inv_s
    denom = p1 + p2 + 1e-9
    w_out_ref[:, 0:1] = p1 / denom
    w_out_ref[:, 1:2] = p2 / denom
    idx_out_ref[:, 0:1] = idx1
    idx_out_ref[:, 1:2] = idx2

    cnt_ref[:] += jnp.sum(onehot1.astype(jnp.float32), axis=0, keepdims=True)
    psum_ref[:] += jnp.sum(probs, axis=0, keepdims=True)

    @pl.when(i == pl.num_programs(0) - 1)
    def _finalize():
        scale = NUM_EXPERTS / (NUM_TOKENS * NUM_TOKENS)
        aux_ref[0] = scale * jnp.sum(cnt_ref[:] * psum_ref[:])


def kernel(x, W):
    grid = NUM_TOKENS // BLOCK_T
    w_out, idx_out, aux = pl.pallas_call(
        _router_block,
        grid=(grid,),
        in_specs=[
            pl.BlockSpec((BLOCK_T, HIDDEN), lambda i: (i, 0)),
            pl.BlockSpec((NUM_EXPERTS, HIDDEN), lambda i: (0, 0)),
        ],
        out_specs=[
            pl.BlockSpec((BLOCK_T, TOP_K), lambda i: (i, 0)),
            pl.BlockSpec((BLOCK_T, TOP_K), lambda i: (i, 0)),
            pl.BlockSpec(memory_space=pltpu.SMEM),
        ],
        out_shape=[
            jax.ShapeDtypeStruct((NUM_TOKENS, TOP_K), jnp.float32),
            jax.ShapeDtypeStruct((NUM_TOKENS, TOP_K), jnp.int32),
            jax.ShapeDtypeStruct((1,), jnp.float32),
        ],
        scratch_shapes=[
            pltpu.VMEM((1, NUM_EXPERTS), jnp.float32),
            pltpu.VMEM((1, NUM_EXPERTS), jnp.float32),
        ],
    )(x, W)
    return (w_out, idx_out, aux[0])
